# baseline (speedup 1.0000x reference)
import functools

import jax
import jax.numpy as jnp
from jax import lax
from jax.experimental import pallas as pl
from jax.experimental.pallas import tpu as pltpu
from jax.experimental.pallas import tpu_sc as plsc

VOCAB = 1000
EMBED_DIM = 64
BATCH = 4096
HIST = 50

_NC = 2
_NS = 16
_NW = _NC * _NS

_B_PER_W = BATCH // _NW
_NBUF = 8
_GROUPS = _B_PER_W // _NBUF


@functools.partial(
    pl.kernel,
    mesh=plsc.VectorSubcoreMesh(core_axis_name="c", subcore_axis_name="s"),
    out_type=jax.ShapeDtypeStruct((BATCH, HIST, EMBED_DIM), jnp.float32),
    scratch_types=[
        pltpu.VMEM((_B_PER_W, HIST), jnp.int32),
        pltpu.VMEM((_NBUF, HIST, EMBED_DIM), jnp.float32),
    ] + [pltpu.SemaphoreType.DMA] * (2 * _NBUF),
    compiler_params=pltpu.CompilerParams(use_tc_tiling_on_sc=False),
)
def _gather_kernel(x_hbm, table_hbm, out_hbm, idx_v, rows_v, *sems):
    gsems = sems[:_NBUF]
    wsems = sems[_NBUF:]
    wid = lax.axis_index("s") * _NC + lax.axis_index("c")
    base = wid * _B_PER_W

    pltpu.sync_copy(x_hbm.at[pl.ds(base, _B_PER_W)], idx_v)

    def body(g, carry):
        for u in range(_NBUF):
            c = g * _NBUF + u

            @pl.when(g > 0)
            def _():
                pltpu.make_async_copy(
                    rows_v.at[u], out_hbm.at[base], wsems[u]
                ).wait()

            pltpu.async_copy(
                table_hbm.at[idx_v.at[c]], rows_v.at[u], gsems[u]
            )
        for u in range(_NBUF):
            c = g * _NBUF + u
            pltpu.make_async_copy(
                table_hbm.at[idx_v.at[c]], rows_v.at[u], gsems[u]
            ).wait()
            pltpu.async_copy(rows_v.at[u], out_hbm.at[base + c], wsems[u])
        return carry

    lax.fori_loop(0, _GROUPS, body, 0)

    for u in range(_NBUF):
        pltpu.make_async_copy(rows_v.at[u], out_hbm.at[base], wsems[u]).wait()


def kernel(x, table):
    return _gather_kernel(x, table)
